# baseline (speedup 1.0000x reference)
import functools

import jax
import jax.numpy as jnp
import numpy as np
from jax import lax
from jax.experimental import pallas as pl
from jax.experimental.pallas import tpu as pltpu
from jax.experimental.pallas import tpu_sc as plsc

FEATS = 1000000
F = 26
D = 16
B = 4096
NPAIR = (F * (F - 1)) // 2
OUTW = NPAIR + F * D

NC = 2
NS = 16
NW = NC * NS
BF = B * F
B_PER_W = BF // NW
CHUNK = 128
NCHUNK = B_PER_W // CHUNK

@functools.cache
def _make_sc_gather():
    mesh = plsc.VectorSubcoreMesh(
        core_axis_name="c", subcore_axis_name="s", num_cores=NC, num_subcores=NS
    )

    @functools.partial(
        pl.kernel,
        out_type=jax.ShapeDtypeStruct((BF, D), jnp.float32),
        mesh=mesh,
        scratch_types=[
            pltpu.VMEM((B_PER_W,), jnp.int32),
            pltpu.VMEM((B_PER_W, D), jnp.float32),
            pltpu.SemaphoreType.DMA,
        ],
        compiler_params=pltpu.CompilerParams(use_tc_tiling_on_sc=False),
    )
    def _sc_gather(idx_hbm, table_hbm, out_hbm, idx_v, rows_v, sem):
        wid = lax.axis_index("s") * NC + lax.axis_index("c")
        base = wid * B_PER_W
        pltpu.sync_copy(idx_hbm.at[pl.ds(base, B_PER_W)], idx_v)
        copies = []
        for c in range(NCHUNK):
            copies.append(
                pltpu.async_copy(
                    table_hbm.at[idx_v.at[pl.ds(c * CHUNK, CHUNK)]],
                    rows_v.at[pl.ds(c * CHUNK, CHUNK)],
                    sem,
                )
            )
        for cp in copies:
            cp.wait()
        pltpu.sync_copy(rows_v, out_hbm.at[pl.ds(base, B_PER_W)])

    return _sc_gather


def _pair_body(emb_ref, k_ref, out_ref):
    e = emb_ref[...]
    kmat = k_ref[...]
    parts = []
    for i in range(F - 1):
        w = F - 1 - i
        ei = e[:, D * i : D * (i + 1)]
        tiled = jnp.concatenate([ei] * w, axis=1)
        rest = e[:, D * (i + 1) :]
        prod = tiled * rest
        red = lax.dot_general(
            prod,
            kmat[: w * D, :w],
            (((1,), (0,)), ((), ())),
            preferred_element_type=jnp.float32,
        )
        parts.append(red)
    parts.append(e)
    out_ref[...] = jnp.concatenate(parts, axis=1)


_BT = 256


def _tc_pairwise(emb2, kmat):
    return pl.pallas_call(
        _pair_body,
        grid=(B // _BT,),
        in_specs=[
            pl.BlockSpec((_BT, F * D), lambda i: (i, 0)),
            pl.BlockSpec((F * D, F), lambda i: (0, 0)),
        ],
        out_specs=pl.BlockSpec((_BT, OUTW), lambda i: (i, 0)),
        out_shape=jax.ShapeDtypeStruct((B, OUTW), jnp.float32),
    )(emb2, kmat)


_K_NP = np.zeros((F * D, F), dtype=np.float32)
for _f in range(F):
    _K_NP[_f * D : (_f + 1) * D, _f] = 1.0


def kernel(x, table):
    idx = x.reshape(BF)
    emb = _make_sc_gather()(idx, table)
    emb2 = emb.reshape(B, F * D)
    return _tc_pairwise(emb2, jnp.asarray(_K_NP))


# speedup vs baseline: 1.0188x; 1.0188x over previous
import functools

import jax
import jax.numpy as jnp
import numpy as np
from jax import lax
from jax.experimental import pallas as pl
from jax.experimental.pallas import tpu as pltpu
from jax.experimental.pallas import tpu_sc as plsc

FEATS = 1000000
F = 26
D = 16
B = 4096
NPAIR = (F * (F - 1)) // 2
OUTW = NPAIR + F * D

NC = 2
NS = 16
NW = NC * NS
BF = B * F
B_PER_W = BF // NW
CHUNK = 128
NCHUNK = B_PER_W // CHUNK

@functools.cache
def _make_sc_gather():
    mesh = plsc.VectorSubcoreMesh(
        core_axis_name="c", subcore_axis_name="s", num_cores=NC, num_subcores=NS
    )

    @functools.partial(
        pl.kernel,
        out_type=jax.ShapeDtypeStruct((BF, D), jnp.float32),
        mesh=mesh,
        scratch_types=[
            pltpu.VMEM((B_PER_W,), jnp.int32),
            pltpu.VMEM((B_PER_W, D), jnp.float32),
            pltpu.SemaphoreType.DMA,
        ],
        compiler_params=pltpu.CompilerParams(use_tc_tiling_on_sc=False),
    )
    def _sc_gather(idx_hbm, table_hbm, out_hbm, idx_v, rows_v, sem):
        wid = lax.axis_index("s") * NC + lax.axis_index("c")
        base = wid * B_PER_W
        pltpu.sync_copy(idx_hbm.at[pl.ds(base, B_PER_W)], idx_v)
        copies = []
        for c in range(NCHUNK):
            copies.append(
                pltpu.async_copy(
                    table_hbm.at[idx_v.at[pl.ds(c * CHUNK, CHUNK)]],
                    rows_v.at[pl.ds(c * CHUNK, CHUNK)],
                    sem,
                )
            )
        for cp in copies:
            cp.wait()
        pltpu.sync_copy(rows_v, out_hbm.at[pl.ds(base, B_PER_W)])

    return _sc_gather


def _pair_body(emb_ref, k_ref, out_ref):
    e = emb_ref[...]
    kmat = k_ref[...]
    parts = []
    for i in range(F - 1):
        w = F - 1 - i
        ei = e[:, D * i : D * (i + 1)]
        tiled = jnp.concatenate([ei] * w, axis=1)
        rest = e[:, D * (i + 1) :]
        prod = tiled * rest
        red = lax.dot_general(
            prod,
            kmat[: w * D, :w],
            (((1,), (0,)), ((), ())),
            preferred_element_type=jnp.float32,
        )
        parts.append(red)
    parts.append(e)
    full = jnp.concatenate(parts, axis=1)
    out_ref[...] = full.T


_BT = 256


def _tc_pairwise(emb2, kmat):
    return pl.pallas_call(
        _pair_body,
        grid=(B // _BT,),
        in_specs=[
            pl.BlockSpec((_BT, F * D), lambda i: (i, 0)),
            pl.BlockSpec((F * D, F), lambda i: (0, 0)),
        ],
        out_specs=pl.BlockSpec((OUTW, _BT), lambda i: (0, i)),
        out_shape=jax.ShapeDtypeStruct((OUTW, B), jnp.float32),
    )(emb2, kmat)


_K_NP = np.zeros((F * D, F), dtype=np.float32)
for _f in range(F):
    _K_NP[_f * D : (_f + 1) * D, _f] = 1.0


def kernel(x, table):
    idx = x.reshape(BF)
    emb = _make_sc_gather()(idx, table)
    emb2 = emb.reshape(B, F * D)
    return _tc_pairwise(emb2, jnp.asarray(_K_NP)).T


# speedup vs baseline: 1.1228x; 1.1021x over previous
import functools

import jax
import jax.numpy as jnp
import numpy as np
from jax import lax
from jax.experimental import pallas as pl
from jax.experimental.pallas import tpu as pltpu
from jax.experimental.pallas import tpu_sc as plsc

FEATS = 1000000
F = 26
D = 16
B = 4096
NPAIR = (F * (F - 1)) // 2
OUTW = NPAIR + F * D

NC = 2
NS = 16
NW = NC * NS
BF = B * F

RW = 32768
CW = 2048
NCHUNK_R = RW // CW
CAP = 4096
XP = 4096
NPIECE = BF // XP
REGCAP = BF + 2048
NSCR = NW * REGCAP
TRASH = BF
TAIL0 = 999424
TAILB = 999936


def _take(a, i):
    return lax.gather(
        a,
        i[:, None],
        dimension_numbers=lax.GatherDimensionNumbers(
            offset_dims=(), collapsed_slice_dims=(0,), start_index_map=(0,)
        ),
        slice_sizes=(1,),
        mode=lax.GatherScatterMode.PROMISE_IN_BOUNDS,
    )


@functools.cache
def _make_sc_extract():
    mesh = plsc.VectorSubcoreMesh(
        core_axis_name="c", subcore_axis_name="s", num_cores=NC, num_subcores=NS
    )

    @functools.partial(
        pl.kernel,
        out_type=(
            jax.ShapeDtypeStruct((NSCR // 8, 128), jnp.float32),
            jax.ShapeDtypeStruct((NSCR,), jnp.int32),
            jax.ShapeDtypeStruct((NW * 16,), jnp.int32),
        ),
        mesh=mesh,
        scratch_types=[
            pltpu.VMEM((XP,), jnp.int32),
            pltpu.VMEM((CAP + 1216,), jnp.int32),
            pltpu.VMEM((CAP + 1216,), jnp.int32),
            pltpu.VMEM((CAP + 1216,), jnp.int32),
            pltpu.VMEM((CAP + 1216,), jnp.int32),
            pltpu.VMEM((16,), jnp.int32),
            pltpu.VMEM((16,), jnp.int32),
            pltpu.VMEM((D, CW), jnp.float32),
            pltpu.VMEM((8, 128), jnp.float32),
            pltpu.VMEM((64,), jnp.int32),
        ],
        compiler_params=pltpu.CompilerParams(
            use_tc_tiling_on_sc=True, needs_layout_passes=False
        ),
    )
    def b1(idx_hbm, tt_hbm, tail_hbm, scr_hbm, jl_hbm, end_hbm,
           idxbuf, p1r, p1j, p2r, p2j, histv, offv, slab, stage, stagej):
        wid = lax.axis_index("s") * NC + lax.axis_index("c")
        lo = wid * RW
        hi = lo + RW
        regbase = wid * REGCAP
        iota = lax.iota(jnp.int32, 16)
        sent_j = jnp.full((16,), TRASH, jnp.int32)

        def round_body(rnd, endpos_in):
            def piece(p, sc):
                def vreg(v, sc2):
                    cnt2, tot2 = sc2
                    r = idxbuf[pl.ds(v * 16, 16)]
                    m = (r >= lo) & (r < hi)
                    mi = m.astype(jnp.int32)
                    pr = plsc.cumsum(mi)
                    ordv = tot2 + pr - 1
                    win = m & (ordv >= rnd * CAP) & (ordv < (rnd + 1) * CAP)
                    jv = p * XP + v * 16 + iota
                    plsc.store_compressed(p1r.at[pl.ds(cnt2, 16)], r, mask=win)
                    plsc.store_compressed(p1j.at[pl.ds(cnt2, 16)], jv, mask=win)
                    return (cnt2 + jnp.sum(win.astype(jnp.int32)),
                            tot2 + jnp.sum(mi))

                pltpu.sync_copy(
                    idx_hbm.at[pl.ds(pl.multiple_of(p * XP, 8), XP)], idxbuf)
                return lax.fori_loop(0, XP // 16, vreg, sc)

            cnt, total = lax.fori_loop(0, NPIECE, piece, (0, 0))
            p1r[pl.ds(cnt, 16)] = jnp.full((16,), 0, jnp.int32) + lo
            p1j[pl.ds(cnt, 16)] = sent_j
            nv = (cnt + 15) // 16

            histv[pl.ds(0, 16)] = jnp.zeros((16,), jnp.int32)

            def c1(g, _):
                r = p1r[pl.ds(g * 16, 16)]
                c = lax.shift_right_logical(r - lo, 11)
                ks, _ls = plsc.sort_key_val(c * 16 + iota, iota)
                cs = lax.shift_right_logical(ks, 4)
                prv = _take(cs, jnp.maximum(iota - 1, 0))
                is_start = (iota == 0) | (cs != prv)
                startpos = plsc.cummax(jnp.where(is_start, iota, 0))
                nxt = _take(cs, jnp.minimum(iota + 1, 15))
                is_end = (iota == 15) | (cs != nxt)
                plsc.addupdate_scatter(histv, [cs], iota - startpos + 1,
                                       mask=is_end)
                return 0

            lax.fori_loop(0, nv, c1, 0)
            h = histv[pl.ds(0, 16)]
            h64 = ((h + 63) // 64) * 64
            hcum = plsc.cumsum(h64)
            seg = hcum - h64
            offv[pl.ds(0, 16)] = seg
            placed = jnp.sum(h64)

            def sfill(g, _):
                p2r[pl.ds(g * 16, 16)] = jnp.full((16,), 0, jnp.int32) + lo
                p2j[pl.ds(g * 16, 16)] = sent_j
                return 0

            lax.fori_loop(0, placed // 16 + 4, sfill, 0)

            def c2(g, _):
                r = p1r[pl.ds(g * 16, 16)]
                jv = p1j[pl.ds(g * 16, 16)]
                c = lax.shift_right_logical(r - lo, 11)
                ks, ls = plsc.sort_key_val(c * 16 + iota, iota)
                cs = lax.shift_right_logical(ks, 4)
                rs = _take(r, ls)
                js = _take(jv, ls)
                prv = _take(cs, jnp.maximum(iota - 1, 0))
                is_start = (iota == 0) | (cs != prv)
                startpos = plsc.cummax(jnp.where(is_start, iota, 0))
                basec = plsc.load_gather(offv, [cs])
                dest = basec + (iota - startpos)
                plsc.store_scatter(p2r, [dest], rs)
                plsc.store_scatter(p2j, [dest], js)
                nxt = _take(cs, jnp.minimum(iota + 1, 15))
                is_end = (iota == 15) | (cs != nxt)
                plsc.store_scatter(offv, [cs], dest + 1, mask=is_end)
                return 0

            lax.fori_loop(0, nv, c2, 0)

            rbase = endpos_in

            def chunk_body(c, _):
                col0 = lo + c * CW

                @pl.when(col0 < FEATS)
                def _chunk():
                    full_w = col0 + CW <= FEATS

                    @pl.when(full_w)
                    def _():
                        pltpu.sync_copy(
                            tt_hbm.at[:, pl.ds(pl.multiple_of(col0, 128), CW)],
                            slab)

                    @pl.when(jnp.logical_not(full_w))
                    def _():
                        pltpu.sync_copy(tt_hbm.at[:, pl.ds(TAIL0, 512)],
                                        slab.at[:, pl.ds(0, 512)])
                        pltpu.sync_copy(tail_hbm, slab.at[:, pl.ds(512, 128)])

                    s0 = jnp.sum(jnp.where(iota == c, seg, 0))
                    send = jnp.where(
                        c == NCHUNK_R - 1,
                        placed,
                        jnp.sum(jnp.where(iota == c + 1, seg, 0)))
                    cstart = jnp.where(full_w, col0, TAIL0)

                    def eg(g, _):
                        p0 = s0 + g * 64
                        for k in range(4):
                            rv = p2r[pl.ds(p0 + k * 16, 16)]
                            jvv = p2j[pl.ds(p0 + k * 16, 16)]
                            colv = jnp.where(rv >= TAILB,
                                             rv - TAILB + 512, rv - cstart)
                            colv = jnp.clip(colv, 0, CW - 1)
                            for d in range(D):
                                vals = plsc.load_gather(
                                    slab,
                                    [jnp.full((16,), d, jnp.int32), colv])
                                flat = k * 256 + iota * 16 + d
                                plsc.store_scatter(
                                    stage,
                                    [lax.shift_right_logical(flat, 7),
                                     flat & 127],
                                    vals)
                            stagej[pl.ds(k * 16, 16)] = jvv
                        pos = rbase + p0
                        pltpu.sync_copy(
                            stage,
                            scr_hbm.at[pl.ds(pl.multiple_of(
                                lax.shift_right_logical(pos, 3), 8), 8), :])
                        pltpu.sync_copy(
                            stagej,
                            jl_hbm.at[pl.ds(pl.multiple_of(pos, 8), 64)])
                        return 0

                    lax.fori_loop(0, (send - s0) // 64, eg, 0)

                return 0

            lax.fori_loop(0, NCHUNK_R, chunk_body, 0)

            return total, endpos_in + placed

        def wbody(carry):
            rnd, _t, ep = carry
            t, ep2 = round_body(rnd, ep)
            return rnd + 1, t, ep2

        def wcond(carry):
            rnd, t, _e = carry
            return jnp.logical_or(rnd == 0, rnd * CAP < t)

        _r, total, endpos = lax.while_loop(wcond, wbody, (0, 1, regbase))

        fill = ((endpos - regbase + 127) // 128) * 128 + regbase - endpos
        stagej[pl.ds(0, 16)] = sent_j

        def padg(g, _):
            pltpu.sync_copy(
                stagej.at[pl.ds(0, 16)],
                jl_hbm.at[pl.ds(pl.multiple_of(endpos + g * 16, 8), 16)])
            return 0

        lax.fori_loop(0, fill // 16, padg, 0)

        stagej[pl.ds(0, 16)] = jnp.full((16,), 0, jnp.int32) + endpos
        pltpu.sync_copy(stagej.at[pl.ds(0, 16)],
                        end_hbm.at[pl.ds(pl.multiple_of(wid * 16, 8), 16)])

    return b1


@functools.cache
def _make_sc_unscramble():
    mesh = plsc.VectorSubcoreMesh(
        core_axis_name="c", subcore_axis_name="s", num_cores=NC, num_subcores=NS
    )

    @functools.partial(
        pl.kernel,
        out_type=jax.ShapeDtypeStruct((BF + 8, D), jnp.float32),
        mesh=mesh,
        scratch_types=[
            pltpu.VMEM((NW * 16,), jnp.int32),
            pltpu.VMEM((128,), jnp.int32),
            pltpu.VMEM((128, D), jnp.float32),
            pltpu.SemaphoreType.DMA,
        ],
        compiler_params=pltpu.CompilerParams(
            use_tc_tiling_on_sc=False, needs_layout_passes=False
        ),
    )
    def b2(scr16_hbm, jl2_hbm, end_hbm, emb_hbm, endsv, jv, rbuf, sem):
        wid = lax.axis_index("s") * NC + lax.axis_index("c")
        regbase = wid * REGCAP
        pltpu.sync_copy(end_hbm, endsv)
        end = jnp.max(endsv[pl.ds(wid * 16, 16)])
        nb = (end - regbase + 127) // 128

        def batch(g, _):
            pos = regbase + g * 128
            pltpu.sync_copy(
                jl2_hbm.at[lax.shift_right_logical(pos, 7)], jv)
            pltpu.sync_copy(
                scr16_hbm.at[pl.ds(pl.multiple_of(pos, 8), 128), :], rbuf)
            pltpu.async_copy(rbuf, emb_hbm.at[jv], sem).wait()
            return 0

        lax.fori_loop(0, nb, batch, 0)

    return b2


def _pair_body(emb_ref, k_ref, out_ref):
    e = emb_ref[...]
    kmat = k_ref[...]
    parts = []
    for i in range(F - 1):
        w = F - 1 - i
        ei = e[:, D * i : D * (i + 1)]
        tiled = jnp.concatenate([ei] * w, axis=1)
        rest = e[:, D * (i + 1) :]
        prod = tiled * rest
        red = lax.dot_general(
            prod,
            kmat[: w * D, :w],
            (((1,), (0,)), ((), ())),
            preferred_element_type=jnp.float32,
        )
        parts.append(red)
    parts.append(e)
    full = jnp.concatenate(parts, axis=1)
    out_ref[...] = full.T


_BT = 256


def _tc_pairwise(emb2, kmat):
    return pl.pallas_call(
        _pair_body,
        grid=(B // _BT,),
        in_specs=[
            pl.BlockSpec((_BT, F * D), lambda i: (i, 0)),
            pl.BlockSpec((F * D, F), lambda i: (0, 0)),
        ],
        out_specs=pl.BlockSpec((OUTW, _BT), lambda i: (0, i)),
        out_shape=jax.ShapeDtypeStruct((OUTW, B), jnp.float32),
    )(emb2, kmat)


_K_NP = np.zeros((F * D, F), dtype=np.float32)
for _f in range(F):
    _K_NP[_f * D : (_f + 1) * D, _f] = 1.0


def kernel(x, table):
    idx = x.reshape(BF)
    table_t = table.T
    tail = jnp.pad(
        lax.slice(table_t, (0, TAILB), (D, FEATS)),
        ((0, 0), (0, 128 - (FEATS - TAILB))),
    )
    scr, jl, ends = _make_sc_extract()(idx, table_t, tail)
    emb_ext = _make_sc_unscramble()(
        scr.reshape(NSCR, D), jl.reshape(NSCR // 128, 128), ends
    )
    emb2 = emb_ext[:BF].reshape(B, F * D)
    return _tc_pairwise(emb2, jnp.asarray(_K_NP)).T


# speedup vs baseline: 1.1658x; 1.0383x over previous
import functools

import jax
import jax.numpy as jnp
import numpy as np
from jax import lax
from jax.experimental import pallas as pl
from jax.experimental.pallas import tpu as pltpu
from jax.experimental.pallas import tpu_sc as plsc

FEATS = 1000000
F = 26
D = 16
B = 4096
NPAIR = (F * (F - 1)) // 2
OUTW = NPAIR + F * D

NC = 2
NS = 16
NW = NC * NS
BF = B * F

RW = 32768
CW = 2048
NCHUNK_R = RW // CW
CAP = 4096
XP = 4096
NPIECE = BF // XP
REGCAP = BF + 2048
NSCR = NW * REGCAP
TRASH = BF
TAIL0 = 999424
TAILB = 999936


def _take(a, i):
    return lax.gather(
        a,
        i[:, None],
        dimension_numbers=lax.GatherDimensionNumbers(
            offset_dims=(), collapsed_slice_dims=(0,), start_index_map=(0,)
        ),
        slice_sizes=(1,),
        mode=lax.GatherScatterMode.PROMISE_IN_BOUNDS,
    )


@functools.cache
def _make_sc_extract():
    mesh = plsc.VectorSubcoreMesh(
        core_axis_name="c", subcore_axis_name="s", num_cores=NC, num_subcores=NS
    )

    @functools.partial(
        pl.kernel,
        out_type=(
            jax.ShapeDtypeStruct((NSCR // 8, 128), jnp.float32),
            jax.ShapeDtypeStruct((NSCR,), jnp.int32),
            jax.ShapeDtypeStruct((NW * 16,), jnp.int32),
        ),
        mesh=mesh,
        scratch_types=[
            pltpu.VMEM((XP,), jnp.int32),
            pltpu.VMEM((CAP + 1216,), jnp.int32),
            pltpu.VMEM((CAP + 1216,), jnp.int32),
            pltpu.VMEM((CAP + 1216,), jnp.int32),
            pltpu.VMEM((CAP + 1216,), jnp.int32),
            pltpu.VMEM((16,), jnp.int32),
            pltpu.VMEM((16,), jnp.int32),
            pltpu.VMEM((D, CW), jnp.float32),
            pltpu.VMEM((8, 128), jnp.float32),
            pltpu.VMEM((64,), jnp.int32),
        ],
        compiler_params=pltpu.CompilerParams(
            use_tc_tiling_on_sc=True, needs_layout_passes=False
        ),
    )
    def b1(idx_hbm, tt_hbm, tail_hbm, scr_hbm, jl_hbm, end_hbm,
           idxbuf, p1r, p1j, p2r, p2j, histv, offv, slab, stage, stagej):
        wid = lax.axis_index("s") * NC + lax.axis_index("c")
        lo = wid * RW
        hi = lo + RW
        regbase = wid * REGCAP
        iota = lax.iota(jnp.int32, 16)
        sent_j = jnp.full((16,), TRASH, jnp.int32)

        def round_body(rnd, endpos_in):
            WCAP = CAP - 64

            def piece(p, sc):
                def vblk(v4, sc2):
                    cnt2, tot2 = sc2
                    accept = (tot2 >= rnd * WCAP) & (tot2 < (rnd + 1) * WCAP)
                    for k in range(4):
                        r = idxbuf[pl.ds(v4 * 64 + k * 16, 16)]
                        m = (r >= lo) & (r < hi)
                        win = m & accept
                        jv = p * XP + v4 * 64 + k * 16 + iota
                        plsc.store_compressed(p1r.at[pl.ds(cnt2, 16)], r,
                                              mask=win)
                        plsc.store_compressed(p1j.at[pl.ds(cnt2, 16)], jv,
                                              mask=win)
                        c_m = plsc.all_reduce_population_count(m)[0]
                        cnt2 = cnt2 + jnp.where(accept, c_m, 0)
                        tot2 = tot2 + c_m
                    return cnt2, tot2

                pltpu.sync_copy(
                    idx_hbm.at[pl.ds(pl.multiple_of(p * XP, 8), XP)], idxbuf)
                return lax.fori_loop(0, XP // 64, vblk, sc)

            cnt, total = lax.fori_loop(0, NPIECE, piece, (0, 0))
            p1r[pl.ds(cnt, 16)] = jnp.full((16,), 0, jnp.int32) + lo
            p1j[pl.ds(cnt, 16)] = sent_j
            nv = (cnt + 15) // 16

            histv[pl.ds(0, 16)] = jnp.zeros((16,), jnp.int32)

            def c1(g, _):
                r = p1r[pl.ds(g * 16, 16)]
                c = lax.shift_right_logical(r - lo, 11)
                ks, _ls = plsc.sort_key_val(c * 16 + iota, iota)
                cs = lax.shift_right_logical(ks, 4)
                prv = _take(cs, jnp.maximum(iota - 1, 0))
                is_start = (iota == 0) | (cs != prv)
                startpos = plsc.cummax(jnp.where(is_start, iota, 0))
                nxt = _take(cs, jnp.minimum(iota + 1, 15))
                is_end = (iota == 15) | (cs != nxt)
                plsc.addupdate_scatter(histv, [cs], iota - startpos + 1,
                                       mask=is_end)
                return 0

            lax.fori_loop(0, nv, c1, 0)
            h = histv[pl.ds(0, 16)]
            h64 = ((h + 63) // 64) * 64
            hcum = plsc.cumsum(h64)
            seg = hcum - h64
            offv[pl.ds(0, 16)] = seg
            placed = jnp.sum(h64)

            def sfill(g, _):
                p2r[pl.ds(g * 16, 16)] = jnp.full((16,), 0, jnp.int32) + lo
                p2j[pl.ds(g * 16, 16)] = sent_j
                return 0

            lax.fori_loop(0, placed // 16 + 4, sfill, 0)

            def c2(g, _):
                r = p1r[pl.ds(g * 16, 16)]
                jv = p1j[pl.ds(g * 16, 16)]
                c = lax.shift_right_logical(r - lo, 11)
                ks, ls = plsc.sort_key_val(c * 16 + iota, iota)
                cs = lax.shift_right_logical(ks, 4)
                rs = _take(r, ls)
                js = _take(jv, ls)
                prv = _take(cs, jnp.maximum(iota - 1, 0))
                is_start = (iota == 0) | (cs != prv)
                startpos = plsc.cummax(jnp.where(is_start, iota, 0))
                basec = plsc.load_gather(offv, [cs])
                dest = basec + (iota - startpos)
                plsc.store_scatter(p2r, [dest], rs)
                plsc.store_scatter(p2j, [dest], js)
                nxt = _take(cs, jnp.minimum(iota + 1, 15))
                is_end = (iota == 15) | (cs != nxt)
                plsc.store_scatter(offv, [cs], dest + 1, mask=is_end)
                return 0

            lax.fori_loop(0, nv, c2, 0)

            rbase = endpos_in

            def chunk_body(c, _):
                col0 = lo + c * CW

                @pl.when(col0 < FEATS)
                def _chunk():
                    full_w = col0 + CW <= FEATS

                    @pl.when(full_w)
                    def _():
                        pltpu.sync_copy(
                            tt_hbm.at[:, pl.ds(pl.multiple_of(col0, 128), CW)],
                            slab)

                    @pl.when(jnp.logical_not(full_w))
                    def _():
                        pltpu.sync_copy(tt_hbm.at[:, pl.ds(TAIL0, 512)],
                                        slab.at[:, pl.ds(0, 512)])
                        pltpu.sync_copy(tail_hbm, slab.at[:, pl.ds(512, 128)])

                    s0 = jnp.sum(jnp.where(iota == c, seg, 0))
                    send = jnp.where(
                        c == NCHUNK_R - 1,
                        placed,
                        jnp.sum(jnp.where(iota == c + 1, seg, 0)))
                    cstart = jnp.where(full_w, col0, TAIL0)

                    def eg(g, _):
                        p0 = s0 + g * 64
                        for k in range(4):
                            rv = p2r[pl.ds(p0 + k * 16, 16)]
                            jvv = p2j[pl.ds(p0 + k * 16, 16)]
                            colv = jnp.where(rv >= TAILB,
                                             rv - TAILB + 512, rv - cstart)
                            colv = jnp.clip(colv, 0, CW - 1)
                            for d in range(D):
                                vals = plsc.load_gather(
                                    slab,
                                    [jnp.full((16,), d, jnp.int32), colv])
                                flat = k * 256 + iota * 16 + d
                                plsc.store_scatter(
                                    stage,
                                    [lax.shift_right_logical(flat, 7),
                                     flat & 127],
                                    vals)
                            stagej[pl.ds(k * 16, 16)] = jvv
                        pos = rbase + p0
                        pltpu.sync_copy(
                            stage,
                            scr_hbm.at[pl.ds(pl.multiple_of(
                                lax.shift_right_logical(pos, 3), 8), 8), :])
                        pltpu.sync_copy(
                            stagej,
                            jl_hbm.at[pl.ds(pl.multiple_of(pos, 8), 64)])
                        return 0

                    lax.fori_loop(0, (send - s0) // 64, eg, 0)

                return 0

            lax.fori_loop(0, NCHUNK_R, chunk_body, 0)

            return total, endpos_in + placed

        def wbody(carry):
            rnd, _t, ep = carry
            t, ep2 = round_body(rnd, ep)
            return rnd + 1, t, ep2

        def wcond(carry):
            rnd, t, _e = carry
            return jnp.logical_or(rnd == 0, rnd * (CAP - 64) < t)

        _r, total, endpos = lax.while_loop(wcond, wbody, (0, 1, regbase))

        fill = ((endpos - regbase + 127) // 128) * 128 + regbase - endpos
        stagej[pl.ds(0, 16)] = sent_j

        def padg(g, _):
            pltpu.sync_copy(
                stagej.at[pl.ds(0, 16)],
                jl_hbm.at[pl.ds(pl.multiple_of(endpos + g * 16, 8), 16)])
            return 0

        lax.fori_loop(0, fill // 16, padg, 0)

        stagej[pl.ds(0, 16)] = jnp.full((16,), 0, jnp.int32) + endpos
        pltpu.sync_copy(stagej.at[pl.ds(0, 16)],
                        end_hbm.at[pl.ds(pl.multiple_of(wid * 16, 8), 16)])

    return b1


@functools.cache
def _make_sc_unscramble():
    mesh = plsc.VectorSubcoreMesh(
        core_axis_name="c", subcore_axis_name="s", num_cores=NC, num_subcores=NS
    )

    @functools.partial(
        pl.kernel,
        out_type=jax.ShapeDtypeStruct((BF + 8, D), jnp.float32),
        mesh=mesh,
        scratch_types=[
            pltpu.VMEM((NW * 16,), jnp.int32),
            pltpu.VMEM((128,), jnp.int32),
            pltpu.VMEM((128,), jnp.int32),
            pltpu.VMEM((16, 128), jnp.float32),
            pltpu.VMEM((16, 128), jnp.float32),
            pltpu.VMEM((128, D), jnp.float32),
            pltpu.VMEM((128, D), jnp.float32),
            pltpu.SemaphoreType.DMA,
            pltpu.SemaphoreType.DMA,
            pltpu.SemaphoreType.DMA,
            pltpu.SemaphoreType.DMA,
        ],
        compiler_params=pltpu.CompilerParams(
            use_tc_tiling_on_sc=False, needs_layout_passes=False
        ),
    )
    def b2(scr_hbm, jl_hbm, end_hbm, emb_hbm, endsv,
           jv0, jv1, rb0, rb1, r20, r21, semr0, semr1, sems0, sems1):
        wid = lax.axis_index("s") * NC + lax.axis_index("c")
        regbase = wid * REGCAP
        iota = lax.iota(jnp.int32, 16)
        jvs, rbs, r2s = (jv0, jv1), (rb0, rb1), (r20, r21)
        semr, sems = (semr0, semr1), (sems0, sems1)
        pltpu.sync_copy(end_hbm, endsv)
        end = jnp.max(endsv[pl.ds(wid * 16, 16)])
        nb = (end - regbase + 127) // 128

        def read_start(g, k):
            pos = regbase + g * 128
            pltpu.async_copy(
                jl_hbm.at[pl.ds(pl.multiple_of(pos, 8), 128)], jvs[k], semr[k])
            pltpu.async_copy(
                scr_hbm.at[pl.ds(pl.multiple_of(
                    lax.shift_right_logical(pos, 3), 8), 16), :],
                rbs[k], semr[k])

        def read_wait(g, k):
            pos = regbase + g * 128
            pltpu.make_async_copy(
                jl_hbm.at[pl.ds(pl.multiple_of(pos, 8), 128)], jvs[k],
                semr[k]).wait()
            pltpu.make_async_copy(
                scr_hbm.at[pl.ds(pl.multiple_of(
                    lax.shift_right_logical(pos, 3), 8), 16), :],
                rbs[k], semr[k]).wait()

        def scat_wait(k):
            pltpu.make_async_copy(r2s[k], emb_hbm.at[jvs[k]], sems[k]).wait()

        @pl.when(nb > 0)
        def _():
            read_start(0, 0)

        def outer(h, _):
            for k in range(2):
                g2 = 2 * h + k

                @pl.when(g2 < nb)
                def _():
                    read_wait(g2, k)

                    @pl.when(g2 >= 1)
                    def _():
                        scat_wait(1 - k)

                    @pl.when(g2 + 1 < nb)
                    def _():
                        read_start(g2 + 1, 1 - k)

                    def rv(a, _):
                        for b2i in range(8):
                            vals = plsc.load_gather(
                                rbs[k],
                                [jnp.full((16,), a, jnp.int32),
                                 b2i * 16 + iota])
                            plsc.store_scatter(
                                r2s[k],
                                [jnp.full((16,), a * 8 + b2i, jnp.int32),
                                 iota], vals)
                        return 0

                    lax.fori_loop(0, 16, rv, 0)
                    pltpu.async_copy(r2s[k], emb_hbm.at[jvs[k]], sems[k])

            return 0

        lax.fori_loop(0, (nb + 1) // 2, outer, 0)

        lastk = (nb - 1) & 1
        for k in range(2):
            @pl.when(jnp.logical_and(nb >= 1, lastk == k))
            def _():
                scat_wait(k)

    return b2


def _pair_body(emb_ref, k_ref, out_ref):
    e = emb_ref[...]
    kmat = k_ref[...]
    parts = []
    for i in range(F - 1):
        w = F - 1 - i
        ei = e[:, D * i : D * (i + 1)]
        tiled = jnp.concatenate([ei] * w, axis=1)
        rest = e[:, D * (i + 1) :]
        prod = tiled * rest
        red = lax.dot_general(
            prod,
            kmat[: w * D, :w],
            (((1,), (0,)), ((), ())),
            preferred_element_type=jnp.float32,
        )
        parts.append(red)
    parts.append(e)
    full = jnp.concatenate(parts, axis=1)
    out_ref[...] = full.T


_BT = 256


def _tc_pairwise(emb2, kmat):
    return pl.pallas_call(
        _pair_body,
        grid=(B // _BT,),
        in_specs=[
            pl.BlockSpec((_BT, F * D), lambda i: (i, 0)),
            pl.BlockSpec((F * D, F), lambda i: (0, 0)),
        ],
        out_specs=pl.BlockSpec((OUTW, _BT), lambda i: (0, i)),
        out_shape=jax.ShapeDtypeStruct((OUTW, B), jnp.float32),
    )(emb2, kmat)


_K_NP = np.zeros((F * D, F), dtype=np.float32)
for _f in range(F):
    _K_NP[_f * D : (_f + 1) * D, _f] = 1.0


def kernel(x, table):
    idx = x.reshape(BF)
    table_t = table.T
    tail = jnp.pad(
        lax.slice(table_t, (0, TAILB), (D, FEATS)),
        ((0, 0), (0, 128 - (FEATS - TAILB))),
    )
    scr, jl, ends = _make_sc_extract()(idx, table_t, tail)
    emb_ext = _make_sc_unscramble()(scr, jl, ends)
    emb2 = emb_ext[:BF].reshape(B, F * D)
    return _tc_pairwise(emb2, jnp.asarray(_K_NP)).T


# speedup vs baseline: 1.2814x; 1.0991x over previous
import functools

import jax
import jax.numpy as jnp
import numpy as np
from jax import lax
from jax.experimental import pallas as pl
from jax.experimental.pallas import tpu as pltpu
from jax.experimental.pallas import tpu_sc as plsc

FEATS = 1000000
F = 26
D = 16
B = 4096
NPAIR = (F * (F - 1)) // 2
OUTW = NPAIR + F * D

NC = 2
NS = 16
NW = NC * NS
BF = B * F

RW = 32768
CW = 2048
NCHUNK_R = RW // CW
CAP = 4096
XP = 4096
NPIECE = BF // XP
REGCAP = BF + 2048
NSCR = NW * REGCAP
TRASH = BF
TAIL0 = 999424
TAILB = 999936


def _take(a, i):
    return lax.gather(
        a,
        i[:, None],
        dimension_numbers=lax.GatherDimensionNumbers(
            offset_dims=(), collapsed_slice_dims=(0,), start_index_map=(0,)
        ),
        slice_sizes=(1,),
        mode=lax.GatherScatterMode.PROMISE_IN_BOUNDS,
    )


@functools.cache
def _make_sc_extract():
    mesh = plsc.VectorSubcoreMesh(
        core_axis_name="c", subcore_axis_name="s", num_cores=NC, num_subcores=NS
    )

    @functools.partial(
        pl.kernel,
        out_type=(
            jax.ShapeDtypeStruct((NSCR // 8, 128), jnp.float32),
            jax.ShapeDtypeStruct((NSCR,), jnp.int32),
            jax.ShapeDtypeStruct((NW * 16,), jnp.int32),
        ),
        mesh=mesh,
        scratch_types=[
            pltpu.VMEM((XP,), jnp.int32),
            pltpu.VMEM((XP,), jnp.int32),
            pltpu.VMEM((CAP + 1216,), jnp.int32),
            pltpu.VMEM((CAP + 1216,), jnp.int32),
            pltpu.VMEM((CAP + 1216,), jnp.int32),
            pltpu.VMEM((CAP + 1216,), jnp.int32),
            pltpu.VMEM((16,), jnp.int32),
            pltpu.VMEM((16,), jnp.int32),
            pltpu.VMEM((D, CW), jnp.float32),
            pltpu.VMEM((D, CW), jnp.float32),
            pltpu.VMEM((8, 128), jnp.float32),
            pltpu.VMEM((64,), jnp.int32),
            pltpu.SemaphoreType.DMA,
            pltpu.SemaphoreType.DMA,
            pltpu.SemaphoreType.DMA,
            pltpu.SemaphoreType.DMA,
        ],
        compiler_params=pltpu.CompilerParams(
            use_tc_tiling_on_sc=True, needs_layout_passes=False
        ),
    )
    def b1(idx_hbm, tt_hbm, tail_hbm, scr_hbm, jl_hbm, end_hbm,
           idxa, idxb, p1r, p1j, p2r, p2j, histv, offv, slaba, slabb,
           stage, stagej, isema, isemb, ssema, ssemb):
        wid = lax.axis_index("s") * NC + lax.axis_index("c")
        lo = wid * RW
        hi = lo + RW
        regbase = wid * REGCAP
        iota = lax.iota(jnp.int32, 16)
        sent_j = jnp.full((16,), TRASH, jnp.int32)
        idxbufs = (idxa, idxb)
        isems = (isema, isemb)
        slabs = (slaba, slabb)
        ssems = (ssema, ssemb)

        def idx_dma(p, k):
            return pltpu.make_async_copy(
                idx_hbm.at[pl.ds(pl.multiple_of(p * XP, 8), XP)],
                idxbufs[k], isems[k])

        def slab_dma_start(c, k):
            col0 = lo + c * CW
            full_w = col0 + CW <= FEATS

            @pl.when(full_w)
            def _():
                pltpu.async_copy(
                    tt_hbm.at[:, pl.ds(pl.multiple_of(col0, 128), CW)],
                    slabs[k], ssems[k])

            @pl.when(jnp.logical_not(full_w))
            def _():
                pltpu.async_copy(tt_hbm.at[:, pl.ds(TAIL0, 512)],
                                 slabs[k].at[:, pl.ds(0, 512)], ssems[k])
                pltpu.async_copy(tail_hbm, slabs[k].at[:, pl.ds(512, 128)],
                                 ssems[k])

        def slab_dma_wait(c, k):
            col0 = lo + c * CW
            full_w = col0 + CW <= FEATS

            @pl.when(full_w)
            def _():
                pltpu.make_async_copy(
                    tt_hbm.at[:, pl.ds(pl.multiple_of(col0, 128), CW)],
                    slabs[k], ssems[k]).wait()

            @pl.when(jnp.logical_not(full_w))
            def _():
                pltpu.make_async_copy(
                    tt_hbm.at[:, pl.ds(TAIL0, 512)],
                    slabs[k].at[:, pl.ds(0, 512)], ssems[k]).wait()
                pltpu.make_async_copy(
                    tail_hbm, slabs[k].at[:, pl.ds(512, 128)],
                    ssems[k]).wait()

        def round_body(rnd, endpos_in):
            WCAP = CAP - 64

            def piece(p, sc, buf):
                def vblk(v4, sc2):
                    cnt2, tot2 = sc2
                    accept = (tot2 >= rnd * WCAP) & (tot2 < (rnd + 1) * WCAP)
                    for k in range(4):
                        r = buf[pl.ds(v4 * 64 + k * 16, 16)]
                        m = (r >= lo) & (r < hi)
                        win = m & accept
                        jv = p * XP + v4 * 64 + k * 16 + iota
                        plsc.store_compressed(p1r.at[pl.ds(cnt2, 16)], r,
                                              mask=win)
                        plsc.store_compressed(p1j.at[pl.ds(cnt2, 16)], jv,
                                              mask=win)
                        c_m = plsc.all_reduce_population_count(m)[0]
                        cnt2 = cnt2 + jnp.where(accept, c_m, 0)
                        tot2 = tot2 + c_m
                    return cnt2, tot2

                return lax.fori_loop(0, XP // 64, vblk, sc)

            idx_dma(0, 0).start()

            def piece2(h, sc):
                for k in range(2):
                    p = 2 * h + k
                    idx_dma(p, k).wait()

                    @pl.when(p + 1 < NPIECE)
                    def _():
                        idx_dma(p + 1, 1 - k).start()

                    sc = piece(p, sc, idxbufs[k])
                return sc

            cnt, total = lax.fori_loop(0, NPIECE // 2, piece2, (0, 0))
            p1r[pl.ds(cnt, 16)] = jnp.full((16,), 0, jnp.int32) + lo
            p1j[pl.ds(cnt, 16)] = sent_j
            nv = (cnt + 15) // 16

            histv[pl.ds(0, 16)] = jnp.zeros((16,), jnp.int32)

            def c1(g, _):
                r = p1r[pl.ds(g * 16, 16)]
                c = lax.shift_right_logical(r - lo, 11)
                ks, _ls = plsc.sort_key_val(c * 16 + iota, iota)
                cs = lax.shift_right_logical(ks, 4)
                prv = _take(cs, jnp.maximum(iota - 1, 0))
                is_start = (iota == 0) | (cs != prv)
                startpos = plsc.cummax(jnp.where(is_start, iota, 0))
                nxt = _take(cs, jnp.minimum(iota + 1, 15))
                is_end = (iota == 15) | (cs != nxt)
                plsc.addupdate_scatter(histv, [cs], iota - startpos + 1,
                                       mask=is_end)
                return 0

            lax.fori_loop(0, nv, c1, 0)
            h = histv[pl.ds(0, 16)]
            h64 = ((h + 63) // 64) * 64
            hcum = plsc.cumsum(h64)
            seg = hcum - h64
            offv[pl.ds(0, 16)] = seg
            placed = jnp.sum(h64)

            def sfill(g, _):
                p2r[pl.ds(g * 16, 16)] = jnp.full((16,), 0, jnp.int32) + lo
                p2j[pl.ds(g * 16, 16)] = sent_j
                return 0

            lax.fori_loop(0, placed // 16 + 4, sfill, 0)

            def c2(g, _):
                r = p1r[pl.ds(g * 16, 16)]
                jv = p1j[pl.ds(g * 16, 16)]
                c = lax.shift_right_logical(r - lo, 11)
                ks, ls = plsc.sort_key_val(c * 16 + iota, iota)
                cs = lax.shift_right_logical(ks, 4)
                rs = _take(r, ls)
                js = _take(jv, ls)
                prv = _take(cs, jnp.maximum(iota - 1, 0))
                is_start = (iota == 0) | (cs != prv)
                startpos = plsc.cummax(jnp.where(is_start, iota, 0))
                basec = plsc.load_gather(offv, [cs])
                dest = basec + (iota - startpos)
                plsc.store_scatter(p2r, [dest], rs)
                plsc.store_scatter(p2j, [dest], js)
                nxt = _take(cs, jnp.minimum(iota + 1, 15))
                is_end = (iota == 15) | (cs != nxt)
                plsc.store_scatter(offv, [cs], dest + 1, mask=is_end)
                return 0

            lax.fori_loop(0, nv, c2, 0)

            rbase = endpos_in

            def chunk_valid(c):
                return lo + c * CW < FEATS

            @pl.when(chunk_valid(0))
            def _():
                slab_dma_start(0, 0)

            def chunk_body(c, slabk):
                col0 = lo + c * CW

                @pl.when(chunk_valid(c))
                def _chunk():
                    full_w = col0 + CW <= FEATS
                    slab_dma_wait(c, slabk)

                    @pl.when(chunk_valid(c + 1) & (c + 1 < NCHUNK_R))
                    def _():
                        slab_dma_start(c + 1, 1 - slabk)

                    s0 = jnp.sum(jnp.where(iota == c, seg, 0))
                    send = jnp.where(
                        c == NCHUNK_R - 1,
                        placed,
                        jnp.sum(jnp.where(iota == c + 1, seg, 0)))
                    cstart = jnp.where(full_w, col0, TAIL0)
                    slab = slabs[slabk]

                    def eg(g, _):
                        p0 = s0 + g * 64
                        for k in range(4):
                            rv = p2r[pl.ds(p0 + k * 16, 16)]
                            jvv = p2j[pl.ds(p0 + k * 16, 16)]
                            colv = jnp.where(rv >= TAILB,
                                             rv - TAILB + 512, rv - cstart)
                            colv = jnp.clip(colv, 0, CW - 1)
                            for d in range(D):
                                vals = plsc.load_gather(
                                    slab,
                                    [jnp.full((16,), d, jnp.int32), colv])
                                flat = k * 256 + iota * 16 + d
                                plsc.store_scatter(
                                    stage,
                                    [lax.shift_right_logical(flat, 7),
                                     flat & 127],
                                    vals)
                            stagej[pl.ds(k * 16, 16)] = jvv
                        pos = rbase + p0
                        pltpu.sync_copy(
                            stage,
                            scr_hbm.at[pl.ds(pl.multiple_of(
                                lax.shift_right_logical(pos, 3), 8), 8), :])
                        pltpu.sync_copy(
                            stagej,
                            jl_hbm.at[pl.ds(pl.multiple_of(pos, 8), 64)])
                        return 0

                    lax.fori_loop(0, (send - s0) // 64, eg, 0)

            def chunk2(h, _):
                for k in range(2):
                    chunk_body(2 * h + k, k)
                return 0

            lax.fori_loop(0, NCHUNK_R // 2, chunk2, 0)

            return total, endpos_in + placed

        def wbody(carry):
            rnd, _t, ep = carry
            t, ep2 = round_body(rnd, ep)
            return rnd + 1, t, ep2

        def wcond(carry):
            rnd, t, _e = carry
            return jnp.logical_or(rnd == 0, rnd * (CAP - 64) < t)

        _r, total, endpos = lax.while_loop(wcond, wbody, (0, 1, regbase))

        fill = ((endpos - regbase + 511) // 512) * 512 + regbase - endpos
        stagej[pl.ds(0, 16)] = sent_j

        def padg(g, _):
            pltpu.sync_copy(
                stagej.at[pl.ds(0, 16)],
                jl_hbm.at[pl.ds(pl.multiple_of(endpos + g * 16, 8), 16)])
            return 0

        lax.fori_loop(0, fill // 16, padg, 0)

        stagej[pl.ds(0, 16)] = jnp.full((16,), 0, jnp.int32) + endpos
        pltpu.sync_copy(stagej.at[pl.ds(0, 16)],
                        end_hbm.at[pl.ds(pl.multiple_of(wid * 16, 8), 16)])

    return b1


@functools.cache
def _make_sc_unscramble():
    mesh = plsc.VectorSubcoreMesh(
        core_axis_name="c", subcore_axis_name="s", num_cores=NC, num_subcores=NS
    )

    @functools.partial(
        pl.kernel,
        out_type=jax.ShapeDtypeStruct((BF + 8, D), jnp.float32),
        mesh=mesh,
        scratch_types=[
            pltpu.VMEM((NW * 16,), jnp.int32),
            pltpu.VMEM((512,), jnp.int32),
            pltpu.VMEM((512,), jnp.int32),
            pltpu.VMEM((4, 128), jnp.int32),
            pltpu.VMEM((4, 128), jnp.int32),
            pltpu.VMEM((64, 128), jnp.float32),
            pltpu.VMEM((64, 128), jnp.float32),
            pltpu.VMEM((512, D), jnp.float32),
            pltpu.VMEM((512, D), jnp.float32),
            pltpu.SemaphoreType.DMA,
            pltpu.SemaphoreType.DMA,
            pltpu.SemaphoreType.DMA,
            pltpu.SemaphoreType.DMA,
        ],
        compiler_params=pltpu.CompilerParams(
            use_tc_tiling_on_sc=False, needs_layout_passes=False
        ),
    )
    def b2(scr_hbm, jl_hbm, end_hbm, emb_hbm, endsv,
           jv0, jv1, jd0, jd1, rb0, rb1, r20, r21,
           semr0, semr1, sems0, sems1):
        wid = lax.axis_index("s") * NC + lax.axis_index("c")
        regbase = wid * REGCAP
        iota = lax.iota(jnp.int32, 16)
        jvs, jds = (jv0, jv1), (jd0, jd1)
        rbs, r2s = (rb0, rb1), (r20, r21)
        semr, sems = (semr0, semr1), (sems0, sems1)
        pltpu.sync_copy(end_hbm, endsv)
        end = jnp.max(endsv[pl.ds(wid * 16, 16)])
        nb = (end - regbase + 511) // 512

        def read_start(g, k):
            pos = regbase + g * 512
            pltpu.async_copy(
                jl_hbm.at[pl.ds(pl.multiple_of(pos, 8), 512)], jvs[k], semr[k])
            pltpu.async_copy(
                scr_hbm.at[pl.ds(pl.multiple_of(
                    lax.shift_right_logical(pos, 3), 8), 64), :],
                rbs[k], semr[k])

        def read_wait(g, k):
            pos = regbase + g * 512
            pltpu.make_async_copy(
                jl_hbm.at[pl.ds(pl.multiple_of(pos, 8), 512)], jvs[k],
                semr[k]).wait()
            pltpu.make_async_copy(
                scr_hbm.at[pl.ds(pl.multiple_of(
                    lax.shift_right_logical(pos, 3), 8), 64), :],
                rbs[k], semr[k]).wait()

        def scat_start(k):
            for q in range(4):
                pltpu.async_copy(
                    r2s[k].at[pl.ds(q * 128, 128), :],
                    emb_hbm.at[jds[k].at[q]], sems[k])

        def scat_wait(k):
            for q in range(4):
                pltpu.make_async_copy(
                    r2s[k].at[pl.ds(q * 128, 128), :],
                    emb_hbm.at[jds[k].at[q]], sems[k]).wait()

        @pl.when(nb > 0)
        def _():
            read_start(0, 0)

        def outer(h, _):
            for k in range(2):
                g2 = 2 * h + k

                @pl.when(g2 < nb)
                def _():
                    read_wait(g2, k)

                    @pl.when(g2 >= 1)
                    def _():
                        scat_wait(1 - k)

                    @pl.when(g2 + 1 < nb)
                    def _():
                        read_start(g2 + 1, 1 - k)

                    for q in range(4):
                        for li in range(8):
                            jds[k][q, pl.ds(li * 16, 16)] = (
                                jvs[k][pl.ds(q * 128 + li * 16, 16)])

                    def rv(a, _):
                        for b2i in range(8):
                            vals = plsc.load_gather(
                                rbs[k],
                                [jnp.full((16,), a, jnp.int32),
                                 b2i * 16 + iota])
                            plsc.store_scatter(
                                r2s[k],
                                [jnp.full((16,), a * 8 + b2i, jnp.int32),
                                 iota], vals)
                        return 0

                    lax.fori_loop(0, 64, rv, 0)
                    scat_start(k)

            return 0

        lax.fori_loop(0, (nb + 1) // 2, outer, 0)

        lastk = (nb - 1) & 1
        for k in range(2):
            @pl.when(jnp.logical_and(nb >= 1, lastk == k))
            def _():
                scat_wait(k)

    return b2


def _pair_body(emb_ref, k_ref, out_ref):
    e = emb_ref[...]
    kmat = k_ref[...]
    parts = []
    for i in range(F - 1):
        w = F - 1 - i
        ei = e[:, D * i : D * (i + 1)]
        tiled = jnp.concatenate([ei] * w, axis=1)
        rest = e[:, D * (i + 1) :]
        prod = tiled * rest
        red = lax.dot_general(
            prod,
            kmat[: w * D, :w],
            (((1,), (0,)), ((), ())),
            preferred_element_type=jnp.float32,
        )
        parts.append(red)
    parts.append(e)
    full = jnp.concatenate(parts, axis=1)
    out_ref[...] = full.T


_BT = 256


def _tc_pairwise(emb2, kmat):
    return pl.pallas_call(
        _pair_body,
        grid=(B // _BT,),
        in_specs=[
            pl.BlockSpec((_BT, F * D), lambda i: (i, 0)),
            pl.BlockSpec((F * D, F), lambda i: (0, 0)),
        ],
        out_specs=pl.BlockSpec((OUTW, _BT), lambda i: (0, i)),
        out_shape=jax.ShapeDtypeStruct((OUTW, B), jnp.float32),
    )(emb2, kmat)


_K_NP = np.zeros((F * D, F), dtype=np.float32)
for _f in range(F):
    _K_NP[_f * D : (_f + 1) * D, _f] = 1.0


def kernel(x, table):
    idx = x.reshape(BF)
    table_t = table.T
    tail = jnp.pad(
        lax.slice(table_t, (0, TAILB), (D, FEATS)),
        ((0, 0), (0, 128 - (FEATS - TAILB))),
    )
    scr, jl, ends = _make_sc_extract()(idx, table_t, tail)
    emb_ext = _make_sc_unscramble()(scr, jl, ends)
    emb2 = emb_ext[:BF].reshape(B, F * D)
    return _tc_pairwise(emb2, jnp.asarray(_K_NP)).T


# speedup vs baseline: 1.6548x; 1.2915x over previous
import functools

import jax
import jax.numpy as jnp
import numpy as np
from jax import lax
from jax.experimental import pallas as pl
from jax.experimental.pallas import tpu as pltpu
from jax.experimental.pallas import tpu_sc as plsc

FEATS = 1000000
F = 26
D = 16
B = 4096
NPAIR = (F * (F - 1)) // 2
OUTW = NPAIR + F * D

NC = 2
NS = 16
NW = NC * NS
BF = B * F

RW = 32768
CW = 2048
NCHUNK_R = RW // CW
CAP = 4096
XP = 4096
NPIECE = BF // XP
REGCAP = BF + 2048
NSCR = NW * REGCAP
TRASH = BF
TAIL0 = 999424
TAILB = 999936


def _take(a, i):
    return lax.gather(
        a,
        i[:, None],
        dimension_numbers=lax.GatherDimensionNumbers(
            offset_dims=(), collapsed_slice_dims=(0,), start_index_map=(0,)
        ),
        slice_sizes=(1,),
        mode=lax.GatherScatterMode.PROMISE_IN_BOUNDS,
    )


@functools.cache
def _make_sc_extract():
    mesh = plsc.VectorSubcoreMesh(
        core_axis_name="c", subcore_axis_name="s", num_cores=NC, num_subcores=NS
    )

    @functools.partial(
        pl.kernel,
        out_type=(
            jax.ShapeDtypeStruct((NSCR // 8, 128), jnp.float32),
            jax.ShapeDtypeStruct((NSCR,), jnp.int32),
            jax.ShapeDtypeStruct((NW * 16,), jnp.int32),
        ),
        mesh=mesh,
        scratch_types=[
            pltpu.VMEM((XP,), jnp.int32),
            pltpu.VMEM((XP,), jnp.int32),
            pltpu.VMEM((CAP + 1216,), jnp.int32),
            pltpu.VMEM((CAP + 1216,), jnp.int32),
            pltpu.VMEM((CAP + 1216,), jnp.int32),
            pltpu.VMEM((CAP + 1216,), jnp.int32),
            pltpu.VMEM((16,), jnp.int32),
            pltpu.VMEM((16,), jnp.int32),
            pltpu.VMEM((D, CW), jnp.float32),
            pltpu.VMEM((D, CW), jnp.float32),
            pltpu.VMEM((8, 128), jnp.float32),
            pltpu.VMEM((64,), jnp.int32),
            pltpu.SemaphoreType.DMA,
            pltpu.SemaphoreType.DMA,
            pltpu.SemaphoreType.DMA,
            pltpu.SemaphoreType.DMA,
        ],
        compiler_params=pltpu.CompilerParams(
            use_tc_tiling_on_sc=True, needs_layout_passes=False
        ),
    )
    def b1(idx_hbm, tt_hbm, tail_hbm, scr_hbm, jl_hbm, end_hbm,
           idxa, idxb, p1r, p1j, p2r, p2j, histv, offv, slaba, slabb,
           stage, stagej, isema, isemb, ssema, ssemb):
        wid = lax.axis_index("s") * NC + lax.axis_index("c")
        lo = wid * RW
        hi = lo + RW
        regbase = wid * REGCAP
        iota = lax.iota(jnp.int32, 16)
        sent_j = jnp.full((16,), TRASH, jnp.int32)
        idxbufs = (idxa, idxb)
        isems = (isema, isemb)
        slabs = (slaba, slabb)
        ssems = (ssema, ssemb)

        def idx_dma(p, k):
            return pltpu.make_async_copy(
                idx_hbm.at[pl.ds(pl.multiple_of(p * XP, 8), XP)],
                idxbufs[k], isems[k])

        def slab_dma_start(c, k):
            col0 = lo + c * CW
            full_w = col0 + CW <= FEATS

            @pl.when(full_w)
            def _():
                pltpu.async_copy(
                    tt_hbm.at[:, pl.ds(pl.multiple_of(col0, 128), CW)],
                    slabs[k], ssems[k])

            @pl.when(jnp.logical_not(full_w))
            def _():
                pltpu.async_copy(tt_hbm.at[:, pl.ds(TAIL0, 512)],
                                 slabs[k].at[:, pl.ds(0, 512)], ssems[k])
                pltpu.async_copy(tail_hbm, slabs[k].at[:, pl.ds(512, 128)],
                                 ssems[k])

        def slab_dma_wait(c, k):
            col0 = lo + c * CW
            full_w = col0 + CW <= FEATS

            @pl.when(full_w)
            def _():
                pltpu.make_async_copy(
                    tt_hbm.at[:, pl.ds(pl.multiple_of(col0, 128), CW)],
                    slabs[k], ssems[k]).wait()

            @pl.when(jnp.logical_not(full_w))
            def _():
                pltpu.make_async_copy(
                    tt_hbm.at[:, pl.ds(TAIL0, 512)],
                    slabs[k].at[:, pl.ds(0, 512)], ssems[k]).wait()
                pltpu.make_async_copy(
                    tail_hbm, slabs[k].at[:, pl.ds(512, 128)],
                    ssems[k]).wait()

        def round_body(rnd, endpos_in):
            WCAP = CAP - 64

            def piece(p, sc, buf):
                def vblk(v4, sc2):
                    cnt2, tot2 = sc2
                    accept = (tot2 >= rnd * WCAP) & (tot2 < (rnd + 1) * WCAP)
                    for k in range(4):
                        r = buf[pl.ds(v4 * 64 + k * 16, 16)]
                        m = (r >= lo) & (r < hi)
                        win = m & accept
                        jv = p * XP + v4 * 64 + k * 16 + iota
                        plsc.store_compressed(p1r.at[pl.ds(cnt2, 16)], r,
                                              mask=win)
                        plsc.store_compressed(p1j.at[pl.ds(cnt2, 16)], jv,
                                              mask=win)
                        c_m = plsc.all_reduce_population_count(m)[0]
                        cnt2 = cnt2 + jnp.where(accept, c_m, 0)
                        tot2 = tot2 + c_m
                    return cnt2, tot2

                return lax.fori_loop(0, XP // 64, vblk, sc)

            idx_dma(0, 0).start()

            def piece2(h, sc):
                for k in range(2):
                    p = 2 * h + k
                    idx_dma(p, k).wait()

                    @pl.when(p + 1 < NPIECE)
                    def _():
                        idx_dma(p + 1, 1 - k).start()

                    sc = piece(p, sc, idxbufs[k])
                return sc

            cnt, total = lax.fori_loop(0, NPIECE // 2, piece2, (0, 0))
            p1r[pl.ds(cnt, 16)] = jnp.full((16,), 0, jnp.int32) + lo
            p1j[pl.ds(cnt, 16)] = sent_j
            nv = (cnt + 15) // 16

            histv[pl.ds(0, 16)] = jnp.zeros((16,), jnp.int32)

            def c1(g, _):
                r = p1r[pl.ds(g * 16, 16)]
                c = lax.shift_right_logical(r - lo, 11)
                ks, _ls = plsc.sort_key_val(c * 16 + iota, iota)
                cs = lax.shift_right_logical(ks, 4)
                prv = _take(cs, jnp.maximum(iota - 1, 0))
                is_start = (iota == 0) | (cs != prv)
                startpos = plsc.cummax(jnp.where(is_start, iota, 0))
                nxt = _take(cs, jnp.minimum(iota + 1, 15))
                is_end = (iota == 15) | (cs != nxt)
                plsc.addupdate_scatter(histv, [cs], iota - startpos + 1,
                                       mask=is_end)
                return 0

            lax.fori_loop(0, nv, c1, 0)
            h = histv[pl.ds(0, 16)]
            h64 = ((h + 63) // 64) * 64
            hcum = plsc.cumsum(h64)
            seg = hcum - h64
            offv[pl.ds(0, 16)] = seg
            placed = jnp.sum(h64)

            def sfill(g, _):
                p2r[pl.ds(g * 16, 16)] = jnp.full((16,), 0, jnp.int32) + lo
                p2j[pl.ds(g * 16, 16)] = sent_j
                return 0

            lax.fori_loop(0, placed // 16 + 4, sfill, 0)

            def c2(g, _):
                r = p1r[pl.ds(g * 16, 16)]
                jv = p1j[pl.ds(g * 16, 16)]
                c = lax.shift_right_logical(r - lo, 11)
                ks, ls = plsc.sort_key_val(c * 16 + iota, iota)
                cs = lax.shift_right_logical(ks, 4)
                rs = _take(r, ls)
                js = _take(jv, ls)
                prv = _take(cs, jnp.maximum(iota - 1, 0))
                is_start = (iota == 0) | (cs != prv)
                startpos = plsc.cummax(jnp.where(is_start, iota, 0))
                basec = plsc.load_gather(offv, [cs])
                dest = basec + (iota - startpos)
                plsc.store_scatter(p2r, [dest], rs)
                plsc.store_scatter(p2j, [dest], js)
                nxt = _take(cs, jnp.minimum(iota + 1, 15))
                is_end = (iota == 15) | (cs != nxt)
                plsc.store_scatter(offv, [cs], dest + 1, mask=is_end)
                return 0

            lax.fori_loop(0, nv, c2, 0)

            rbase = endpos_in

            def chunk_valid(c):
                return lo + c * CW < FEATS

            @pl.when(chunk_valid(0))
            def _():
                slab_dma_start(0, 0)

            def chunk_body(c, slabk):
                col0 = lo + c * CW

                @pl.when(chunk_valid(c))
                def _chunk():
                    full_w = col0 + CW <= FEATS
                    slab_dma_wait(c, slabk)

                    @pl.when(chunk_valid(c + 1) & (c + 1 < NCHUNK_R))
                    def _():
                        slab_dma_start(c + 1, 1 - slabk)

                    s0 = jnp.sum(jnp.where(iota == c, seg, 0))
                    send = jnp.where(
                        c == NCHUNK_R - 1,
                        placed,
                        jnp.sum(jnp.where(iota == c + 1, seg, 0)))
                    cstart = jnp.where(full_w, col0, TAIL0)
                    slab = slabs[slabk]

                    def eg(g, _):
                        p0 = s0 + g * 64
                        for k in range(4):
                            rv = p2r[pl.ds(p0 + k * 16, 16)]
                            jvv = p2j[pl.ds(p0 + k * 16, 16)]
                            colv = jnp.where(rv >= TAILB,
                                             rv - TAILB + 512, rv - cstart)
                            colv = jnp.clip(colv, 0, CW - 1)
                            for d in range(D):
                                vals = plsc.load_gather(
                                    slab,
                                    [jnp.full((16,), d, jnp.int32), colv])
                                flat = k * 256 + iota * 16 + d
                                plsc.store_scatter(
                                    stage,
                                    [lax.shift_right_logical(flat, 7),
                                     flat & 127],
                                    vals)
                            stagej[pl.ds(k * 16, 16)] = jvv
                        pos = rbase + p0
                        pltpu.sync_copy(
                            stage,
                            scr_hbm.at[pl.ds(pl.multiple_of(
                                lax.shift_right_logical(pos, 3), 8), 8), :])
                        pltpu.sync_copy(
                            stagej,
                            jl_hbm.at[pl.ds(pl.multiple_of(pos, 8), 64)])
                        return 0

                    lax.fori_loop(0, (send - s0) // 64, eg, 0)

            def chunk2(h, _):
                for k in range(2):
                    chunk_body(2 * h + k, k)
                return 0

            lax.fori_loop(0, NCHUNK_R // 2, chunk2, 0)

            return total, endpos_in + placed

        def wbody(carry):
            rnd, _t, ep = carry
            t, ep2 = round_body(rnd, ep)
            return rnd + 1, t, ep2

        def wcond(carry):
            rnd, t, _e = carry
            return jnp.logical_or(rnd == 0, rnd * (CAP - 64) < t)

        _r, total, endpos = lax.while_loop(wcond, wbody, (0, 1, regbase))

        fill = ((endpos - regbase + 511) // 512) * 512 + regbase - endpos
        stagej[pl.ds(0, 16)] = sent_j

        def padg(g, _):
            pltpu.sync_copy(
                stagej.at[pl.ds(0, 16)],
                jl_hbm.at[pl.ds(pl.multiple_of(endpos + g * 16, 8), 16)])
            return 0

        lax.fori_loop(0, fill // 16, padg, 0)

        stagej[pl.ds(0, 16)] = jnp.full((16,), 0, jnp.int32) + endpos
        pltpu.sync_copy(stagej.at[pl.ds(0, 16)],
                        end_hbm.at[pl.ds(pl.multiple_of(wid * 16, 8), 16)])

    return b1


@functools.cache
def _make_sc_unscramble():
    mesh = plsc.VectorSubcoreMesh(
        core_axis_name="c", subcore_axis_name="s", num_cores=NC, num_subcores=NS
    )

    @functools.partial(
        pl.kernel,
        out_type=jax.ShapeDtypeStruct((BF + F, D), jnp.float32),
        mesh=mesh,
        scratch_types=[
            pltpu.VMEM((NW * 16,), jnp.int32),
            pltpu.VMEM((512,), jnp.int32),
            pltpu.VMEM((512,), jnp.int32),
            pltpu.VMEM((4, 128), jnp.int32),
            pltpu.VMEM((4, 128), jnp.int32),
            pltpu.VMEM((64, 128), jnp.float32),
            pltpu.VMEM((64, 128), jnp.float32),
            pltpu.VMEM((512, D), jnp.float32),
            pltpu.VMEM((512, D), jnp.float32),
            pltpu.SemaphoreType.DMA,
            pltpu.SemaphoreType.DMA,
            pltpu.SemaphoreType.DMA,
            pltpu.SemaphoreType.DMA,
        ],
        compiler_params=pltpu.CompilerParams(
            use_tc_tiling_on_sc=False, needs_layout_passes=False
        ),
    )
    def b2(scr_hbm, jl_hbm, end_hbm, emb_hbm, endsv,
           jv0, jv1, jd0, jd1, rb0, rb1, r20, r21,
           semr0, semr1, sems0, sems1):
        wid = lax.axis_index("s") * NC + lax.axis_index("c")
        regbase = wid * REGCAP
        iota = lax.iota(jnp.int32, 16)
        jvs, jds = (jv0, jv1), (jd0, jd1)
        rbs, r2s = (rb0, rb1), (r20, r21)
        semr, sems = (semr0, semr1), (sems0, sems1)
        pltpu.sync_copy(end_hbm, endsv)
        end = jnp.max(endsv[pl.ds(wid * 16, 16)])
        nb = (end - regbase + 511) // 512

        def read_start(g, k):
            pos = regbase + g * 512
            pltpu.async_copy(
                jl_hbm.at[pl.ds(pl.multiple_of(pos, 8), 512)], jvs[k], semr[k])
            pltpu.async_copy(
                scr_hbm.at[pl.ds(pl.multiple_of(
                    lax.shift_right_logical(pos, 3), 8), 64), :],
                rbs[k], semr[k])

        def read_wait(g, k):
            pos = regbase + g * 512
            pltpu.make_async_copy(
                jl_hbm.at[pl.ds(pl.multiple_of(pos, 8), 512)], jvs[k],
                semr[k]).wait()
            pltpu.make_async_copy(
                scr_hbm.at[pl.ds(pl.multiple_of(
                    lax.shift_right_logical(pos, 3), 8), 64), :],
                rbs[k], semr[k]).wait()

        def scat_start(k):
            for q in range(4):
                pltpu.async_copy(
                    r2s[k].at[pl.ds(q * 128, 128), :],
                    emb_hbm.at[jds[k].at[q]], sems[k])

        def scat_wait(k):
            for q in range(4):
                pltpu.make_async_copy(
                    r2s[k].at[pl.ds(q * 128, 128), :],
                    emb_hbm.at[jds[k].at[q]], sems[k]).wait()

        @pl.when(nb > 0)
        def _():
            read_start(0, 0)

        def outer(h, _):
            for k in range(2):
                g2 = 2 * h + k

                @pl.when(g2 < nb)
                def _():
                    read_wait(g2, k)

                    @pl.when(g2 >= 1)
                    def _():
                        scat_wait(1 - k)

                    @pl.when(g2 + 1 < nb)
                    def _():
                        read_start(g2 + 1, 1 - k)

                    for q in range(4):
                        for li in range(8):
                            jds[k][q, pl.ds(li * 16, 16)] = (
                                jvs[k][pl.ds(q * 128 + li * 16, 16)])

                    def rv(a, _):
                        for b2i in range(8):
                            vals = plsc.load_gather(
                                rbs[k],
                                [jnp.full((16,), a, jnp.int32),
                                 b2i * 16 + iota])
                            plsc.store_scatter(
                                r2s[k],
                                [jnp.full((16,), a * 8 + b2i, jnp.int32),
                                 iota], vals)
                        return 0

                    lax.fori_loop(0, 64, rv, 0)
                    scat_start(k)

            return 0

        lax.fori_loop(0, (nb + 1) // 2, outer, 0)

        lastk = (nb - 1) & 1
        for k in range(2):
            @pl.when(jnp.logical_and(nb >= 1, lastk == k))
            def _():
                scat_wait(k)

    return b2


def _pair_body(emb_ref, k_ref, out_ref):
    e = emb_ref[...]
    kmat = k_ref[...]
    parts = []
    for i in range(F - 1):
        w = F - 1 - i
        ei = e[:, D * i : D * (i + 1)]
        tiled = jnp.concatenate([ei] * w, axis=1)
        rest = e[:, D * (i + 1) :]
        prod = tiled * rest
        red = lax.dot_general(
            prod,
            kmat[: w * D, :w],
            (((1,), (0,)), ((), ())),
            preferred_element_type=jnp.float32,
        )
        parts.append(red)
    parts.append(e)
    full = jnp.concatenate(parts, axis=1)
    out_ref[...] = full.T


_BT = 256


def _tc_pairwise(emb2, kmat):
    return pl.pallas_call(
        _pair_body,
        grid=(B // _BT,),
        in_specs=[
            pl.BlockSpec((_BT, F * D), lambda i: (i, 0)),
            pl.BlockSpec((F * D, F), lambda i: (0, 0)),
        ],
        out_specs=pl.BlockSpec((OUTW, _BT), lambda i: (0, i)),
        out_shape=jax.ShapeDtypeStruct((OUTW, B), jnp.float32),
    )(emb2, kmat)


_K_NP = np.zeros((F * D, F), dtype=np.float32)
for _f in range(F):
    _K_NP[_f * D : (_f + 1) * D, _f] = 1.0


def kernel(x, table):
    idx = x.reshape(BF)
    table_t = table.T
    tail = jnp.pad(
        lax.slice(table_t, (0, TAILB), (D, FEATS)),
        ((0, 0), (0, 128 - (FEATS - TAILB))),
    )
    scr, jl, ends = _make_sc_extract()(idx, table_t, tail)
    emb_ext = _make_sc_unscramble()(scr, jl, ends)
    emb2 = emb_ext.reshape(B + 1, F * D)
    return _tc_pairwise(emb2, jnp.asarray(_K_NP)).T
